# baseline (speedup 1.0000x reference)
import functools

import jax
import jax.numpy as jnp
from jax import lax
from jax.experimental import pallas as pl
from jax.experimental.pallas import tpu as pltpu
from jax.experimental.pallas import tpu_sc as plsc

_H = 8
_D = 32
_HD = _H * _D
_HH = _H // 2
_NSC = 2
_NTILE = 16
_LANES = 16
_EC = 128
_FC = 128


def _tc_embed_kernel(x_ref, w_ref, b_ref, acat_ref, h2_ref, s_ref):
    xb = x_ref[...]
    hb = jnp.dot(xb, w_ref[...], preferred_element_type=jnp.float32)
    hb = hb + b_ref[...]
    nb = hb.shape[0]
    h2_ref[...] = hb.reshape(2 * nb, _HD // 2)
    s_ref[...] = jnp.dot(hb, acat_ref[...], preferred_element_type=jnp.float32)


def _tc_ln_kernel(osc_ref, x_ref, g_ref, bt_ref, out_ref):
    o = jnp.concatenate([osc_ref[0], osc_ref[1]], axis=-1) + x_ref[...]
    mean = jnp.mean(o, axis=1, keepdims=True)
    cent = o - mean
    var = jnp.mean(cent * cent, axis=1, keepdims=True)
    out_ref[...] = cent * lax.rsqrt(var + 1e-5) * g_ref[...] + bt_ref[...]


def _sc_edge_kernel(n_nodes, n_edges,
                    h2_hbm, sflat_hbm, srcix_hbm, dstix_hbm,
                    out_hbm, *sc):
    it = iter(sc)
    srcc = [next(it) for _ in range(2)]
    dstc = [next(it) for _ in range(2)]
    idx2 = [next(it) for _ in range(2)]
    ixs = [[next(it) for _ in range(_HH)] for _ in range(2)]
    jxs = [[next(it) for _ in range(_HH)] for _ in range(2)]
    dix = [next(it) for _ in range(2)]
    gss = [[next(it) for _ in range(_HH)] for _ in range(2)]
    gds = [[next(it) for _ in range(_HH)] for _ in range(2)]
    rows = [next(it) for _ in range(2)]
    fin = rows[0]
    gsem = [next(it) for _ in range(2)]
    ssem = [next(it) for _ in range(2)]
    isem = [next(it) for _ in range(2)]
    out_sp = next(it)
    asums = [next(it) for _ in range(_HH)]
    abs_ = gss[0]

    c = lax.axis_index("c")
    s = lax.axis_index("s")
    sbase = n_nodes * 8 * c

    n_chunks_tot = n_edges // _EC
    base_cnt = n_chunks_tot // _NTILE
    n_extra = n_chunks_tot - base_cnt * _NTILE
    start = base_cnt * s + jnp.minimum(s, n_extra)
    count = jnp.where(s < n_extra, base_cnt + 1, base_cnt)
    max_cnt = base_cnt + (1 if n_extra else 0)
    n_pairs = (max_cnt + 2) // 2

    n_full = n_nodes // _FC
    tail = n_nodes - n_full * _FC
    n_rounds = (n_full + 1 + _NTILE - 1) // _NTILE

    zero16 = jnp.zeros((_LANES,), jnp.float32)

    @pl.loop(0, _FC)
    def _(i):
        for jv in range(_HD // 2 // _LANES):
            fin[i, pl.ds(_LANES * jv, _LANES)] = zero16

    @pl.loop(0, n_rounds)
    def _(r):
        j = _NTILE * r + s

        @pl.when(j < n_full)
        def _():
            n0 = _FC * j
            pltpu.sync_copy(fin, out_sp.at[pl.ds(n0, _FC)])
            for h in range(_HH):
                pltpu.sync_copy(fin.at[0], asums[h].at[pl.ds(n0, _FC)])

        if tail:
            @pl.when(j == n_full)
            def _():
                pltpu.sync_copy(fin.at[pl.ds(0, tail)],
                                out_sp.at[pl.ds(n_full * _FC, tail)])
                for h in range(_HH):
                    pltpu.sync_copy(fin.at[0],
                                    asums[h].at[pl.ds(n_full * _FC, _FC)])

    plsc.subcore_barrier()

    def _load_idx(b, k, sync=False):
        srcsl = srcix_hbm.at[pl.ds(_EC * (start + k), _EC)]
        dstsl = dstix_hbm.at[pl.ds(_EC * (start + k), _EC)]
        if sync:
            pltpu.sync_copy(srcsl, srcc[b])
            pltpu.sync_copy(dstsl, dstc[b])
        else:
            pltpu.async_copy(srcsl, srcc[b], isem[b])
            pltpu.async_copy(dstsl, dstc[b], isem[b])

    def _wait_idx(b, k):
        pltpu.make_async_copy(
            srcix_hbm.at[pl.ds(_EC * (start + k), _EC)], srcc[b],
            isem[b]).wait()
        pltpu.make_async_copy(
            dstix_hbm.at[pl.ds(_EC * (start + k), _EC)], dstc[b],
            isem[b]).wait()

    def _prime(b, k):
        for v in range(_EC // _LANES):
            sl = pl.ds(_LANES * v, _LANES)
            sv = srcc[b][sl]
            dv = dstc[b][sl]
            idx2[b][sl] = sv * 2 + c
            dix[b][sl] = dv
            s16 = sbase + sv * 8
            d16 = sbase + dv * 8 + 4
            for h in range(_HH):
                ixs[b][h][sl] = s16 + h
                jxs[b][h][sl] = d16 + h
        pltpu.async_copy(h2_hbm.at[idx2[b]], rows[b], gsem[b])
        for h in range(_HH):
            pltpu.async_copy(sflat_hbm.at[ixs[b][h]], gss[b][h], gsem[b])
            pltpu.async_copy(sflat_hbm.at[jxs[b][h]], gds[b][h], gsem[b])

    def _wait_gathers(b):
        pltpu.make_async_copy(h2_hbm.at[idx2[b]], rows[b], gsem[b]).wait()
        for h in range(_HH):
            pltpu.make_async_copy(
                sflat_hbm.at[ixs[b][h]], gss[b][h], gsem[b]).wait()
            pltpu.make_async_copy(
                sflat_hbm.at[jxs[b][h]], gds[b][h], gsem[b]).wait()

    def _process(b):
        _wait_gathers(b)
        for v in range(_EC // _LANES):
            sl = pl.ds(_LANES * v, _LANES)
            for h in range(_HH):
                a = gss[b][h][sl] + gds[b][h][sl]
                a = jnp.where(a >= 0.0, a, 0.2 * a)
                gds[b][h][sl] = jnp.exp(a)

        @pl.loop(0, _EC // _LANES)
        def _(g):
            ge = g * _LANES
            pr = [gds[b][h][pl.ds(ge, _LANES)] for h in range(_HH)]
            for i in range(_LANES):
                lane = jnp.full((_LANES,), i, jnp.int32)
                for h in range(_HH):
                    w = pr[h][lane]
                    sl0 = pl.ds(_D * h, _LANES)
                    sl1 = pl.ds(_D * h + _LANES, _LANES)
                    rows[b][ge + i, sl0] = rows[b][ge + i, sl0] * w
                    rows[b][ge + i, sl1] = rows[b][ge + i, sl1] * w

        pltpu.async_copy(rows[b], out_sp.at[dix[b]], ssem[b], add=True)
        for h in range(_HH):
            pltpu.async_copy(gds[b][h], asums[h].at[dix[b]], ssem[b],
                             add=True)

    def _wait_scatters(b):
        pltpu.make_async_copy(rows[b], out_sp.at[dix[b]], ssem[b]).wait()
        for h in range(_HH):
            pltpu.make_async_copy(
                gds[b][h], asums[h].at[dix[b]], ssem[b]).wait()

    _load_idx(0, jnp.int32(0), sync=True)
    _prime(0, jnp.int32(0))

    @pl.when(1 < count)
    def _():
        _load_idx(1, jnp.int32(1))

    @pl.loop(0, n_pairs)
    def _(kk):
        for b in (0, 1):
            k = 2 * kk + b
            nb_ = 1 - b

            @pl.when((k >= 1) & (k - 1 < count))
            def _():
                _wait_scatters(nb_)

            @pl.when(k + 1 < count)
            def _():
                _wait_idx(nb_, k + 1)
                _prime(nb_, k + 1)

            @pl.when(k + 2 < count)
            def _():
                _load_idx(b, k + 2)

            @pl.when(k < count)
            def _():
                _process(b)

    plsc.subcore_barrier()

    def _norm_block(n0, nrows):
        for h in range(_HH):
            pltpu.sync_copy(asums[h].at[pl.ds(n0, _FC)], abs_[h])

        @pl.loop(0, nrows // _LANES)
        def _(g):
            i0 = g * _LANES
            rec = [1.0 / (abs_[h][pl.ds(i0, _LANES)] + 1e-8)
                   for h in range(_HH)]
            for i in range(_LANES):
                lane = jnp.full((_LANES,), i, jnp.int32)
                for h in range(_HH):
                    w = rec[h][lane]
                    sl0 = pl.ds(_D * h, _LANES)
                    sl1 = pl.ds(_D * h + _LANES, _LANES)
                    fin[i0 + i, sl0] = fin[i0 + i, sl0] * w
                    fin[i0 + i, sl1] = fin[i0 + i, sl1] * w

    @pl.loop(0, n_rounds)
    def _(r):
        j = _NTILE * r + s

        @pl.when(j < n_full)
        def _():
            n0 = _FC * j
            pltpu.sync_copy(out_sp.at[pl.ds(n0, _FC)], fin)
            _norm_block(n0, _FC)
            pltpu.sync_copy(fin, out_hbm.at[c, pl.ds(n0, _FC)])

        if tail:
            @pl.when(j == n_full)
            def _():
                n0 = n_full * _FC
                pltpu.sync_copy(out_sp.at[pl.ds(n0, tail)],
                                fin.at[pl.ds(0, tail)])
                _norm_block(n0, tail)
                pltpu.sync_copy(fin.at[pl.ds(0, tail)],
                                out_hbm.at[c, pl.ds(n0, tail)])


def kernel(x, edge_index, W, b, a_src, a_dst, ln_gamma, ln_beta):
    n_nodes, nf = x.shape
    n_edges = edge_index.shape[1]
    n_pad = ((n_nodes + _FC - 1) // _FC) * _FC
    max_cnt = n_edges // _EC // _NTILE + 1

    blk = jnp.repeat(jnp.eye(_H, dtype=jnp.float32), _D, axis=0)
    asrc_m = blk * a_src.reshape(_HD, 1)
    adst_m = blk * a_dst.reshape(_HD, 1)
    acat = jnp.concatenate([asrc_m[:, :_HH], adst_m[:, :_HH],
                            asrc_m[:, _HH:], adst_m[:, _HH:]], axis=1)

    nb = 1000
    grid = n_nodes // nb
    h2, sall = pl.pallas_call(
        _tc_embed_kernel,
        grid=(grid,),
        in_specs=[
            pl.BlockSpec((nb, nf), lambda i: (i, 0)),
            pl.BlockSpec((nf, _HD), lambda i: (0, 0)),
            pl.BlockSpec((1, _HD), lambda i: (0, 0)),
            pl.BlockSpec((nf, 16), lambda i: (0, 0)),
        ],
        out_specs=[
            pl.BlockSpec((2 * nb, _HD // 2), lambda i: (i, 0)),
            pl.BlockSpec((nb, 16), lambda i: (i, 0)),
        ],
        out_shape=[
            jax.ShapeDtypeStruct((2 * n_nodes, _HD // 2), jnp.float32),
            jax.ShapeDtypeStruct((n_nodes, 16), jnp.float32),
        ],
    )(x, W, b.reshape(1, _HD), acat)

    sflat = jnp.concatenate([sall[:, :8].reshape(-1),
                             sall[:, 8:].reshape(-1)])
    src_ix = edge_index[0]
    dst_ix = edge_index[1]

    mesh = plsc.VectorSubcoreMesh(
        core_axis_name="c", subcore_axis_name="s",
        num_cores=_NSC, num_subcores=_NTILE)
    scratch = (
        [pltpu.VMEM((_EC,), jnp.int32)] * 4
        + [pltpu.VMEM((_EC,), jnp.int32)] * 2
        + [pltpu.VMEM((_EC,), jnp.int32)] * (2 * _HH)
        + [pltpu.VMEM((_EC,), jnp.int32)] * (2 * _HH)
        + [pltpu.VMEM((_EC,), jnp.int32)] * 2
        + [pltpu.VMEM((_EC,), jnp.float32)] * (2 * _HH)
        + [pltpu.VMEM((_EC,), jnp.float32)] * (2 * _HH)
        + [pltpu.VMEM((_EC, _HD // 2), jnp.float32)] * 2
        + [pltpu.SemaphoreType.DMA] * 6
        + [pltpu.VMEM_SHARED((n_nodes, _HD // 2), jnp.float32)]
        + [pltpu.VMEM_SHARED((n_pad,), jnp.float32)] * _HH
    )
    out_sc = pl.kernel(
        functools.partial(_sc_edge_kernel, n_nodes, n_edges),
        out_type=jax.ShapeDtypeStruct((_NSC, n_nodes, _HD // 2), jnp.float32),
        mesh=mesh,
        scratch_types=scratch,
    )(h2, sflat, src_ix, dst_ix)

    out = pl.pallas_call(
        _tc_ln_kernel,
        grid=(grid,),
        in_specs=[
            pl.BlockSpec((_NSC, nb, _HD // 2), lambda i: (0, i, 0)),
            pl.BlockSpec((nb, nf), lambda i: (i, 0)),
            pl.BlockSpec((1, _HD), lambda i: (0, 0)),
            pl.BlockSpec((1, _HD), lambda i: (0, 0)),
        ],
        out_specs=pl.BlockSpec((nb, nf), lambda i: (i, 0)),
        out_shape=jax.ShapeDtypeStruct((n_nodes, nf), jnp.float32),
    )(out_sc, x, ln_gamma.reshape(1, _HD), ln_beta.reshape(1, _HD))
    return out


# speedup vs baseline: 1.0654x; 1.0654x over previous
import functools

import jax
import jax.numpy as jnp
from jax import lax
from jax.experimental import pallas as pl
from jax.experimental.pallas import tpu as pltpu
from jax.experimental.pallas import tpu_sc as plsc

_H = 8
_D = 32
_HD = _H * _D
_HH = _H // 2
_NSC = 2
_NTILE = 16
_LANES = 16
_EC = 128
_FC = 128


def _tc_embed_kernel(x_ref, w_ref, b_ref, acat_ref, h2_ref, s_ref):
    xb = x_ref[...]
    hb = jnp.dot(xb, w_ref[...], preferred_element_type=jnp.float32)
    hb = hb + b_ref[...]
    nb = hb.shape[0]
    h2_ref[...] = hb.reshape(2 * nb, _HD // 2)
    sv = jnp.dot(hb, acat_ref[...], preferred_element_type=jnp.float32)
    s_ref[0] = sv[:, :8]
    s_ref[1] = sv[:, 8:]


def _tc_ln_kernel(osc_ref, x_ref, g_ref, bt_ref, out_ref):
    o = jnp.concatenate([osc_ref[0], osc_ref[1]], axis=-1) + x_ref[...]
    mean = jnp.mean(o, axis=1, keepdims=True)
    cent = o - mean
    var = jnp.mean(cent * cent, axis=1, keepdims=True)
    out_ref[...] = cent * lax.rsqrt(var + 1e-5) * g_ref[...] + bt_ref[...]


def _sc_edge_kernel(n_nodes, n_edges,
                    h2_hbm, sflat_hbm, ei_hbm,
                    out_hbm, *sc):
    it = iter(sc)
    srcc = [next(it) for _ in range(2)]
    dstc = [next(it) for _ in range(2)]
    idx2 = [next(it) for _ in range(2)]
    ixs = [[next(it) for _ in range(_HH)] for _ in range(2)]
    jxs = [[next(it) for _ in range(_HH)] for _ in range(2)]
    dix = [next(it) for _ in range(2)]
    gss = [[next(it) for _ in range(_HH)] for _ in range(2)]
    gds = [[next(it) for _ in range(_HH)] for _ in range(2)]
    rows = [next(it) for _ in range(2)]
    fin = rows[0]
    gsem = [next(it) for _ in range(2)]
    ssem = [next(it) for _ in range(2)]
    isem = [next(it) for _ in range(2)]
    out_sp = next(it)
    asums = [next(it) for _ in range(_HH)]
    abs_ = gss[0]

    c = lax.axis_index("c")
    s = lax.axis_index("s")
    sbase = n_nodes * 8 * c

    n_chunks_tot = n_edges // _EC
    base_cnt = n_chunks_tot // _NTILE
    n_extra = n_chunks_tot - base_cnt * _NTILE
    start = base_cnt * s + jnp.minimum(s, n_extra)
    count = jnp.where(s < n_extra, base_cnt + 1, base_cnt)
    max_cnt = base_cnt + (1 if n_extra else 0)
    n_pairs = (max_cnt + 2) // 2

    n_full = n_nodes // _FC
    tail = n_nodes - n_full * _FC
    n_rounds = (n_full + 1 + _NTILE - 1) // _NTILE

    zero16 = jnp.zeros((_LANES,), jnp.float32)

    @pl.loop(0, _FC)
    def _(i):
        for jv in range(_HD // 2 // _LANES):
            fin[i, pl.ds(_LANES * jv, _LANES)] = zero16

    @pl.loop(0, n_rounds)
    def _(r):
        j = _NTILE * r + s

        @pl.when(j < n_full)
        def _():
            n0 = _FC * j
            pltpu.sync_copy(fin, out_sp.at[pl.ds(n0, _FC)])
            for h in range(_HH):
                pltpu.sync_copy(fin.at[0], asums[h].at[pl.ds(n0, _FC)])

        if tail:
            @pl.when(j == n_full)
            def _():
                pltpu.sync_copy(fin.at[pl.ds(0, tail)],
                                out_sp.at[pl.ds(n_full * _FC, tail)])
                for h in range(_HH):
                    pltpu.sync_copy(fin.at[0],
                                    asums[h].at[pl.ds(n_full * _FC, _FC)])

    plsc.subcore_barrier()

    def _load_idx(b, k, sync=False):
        srcsl = ei_hbm.at[0, pl.ds(_EC * (start + k), _EC)]
        dstsl = ei_hbm.at[1, pl.ds(_EC * (start + k), _EC)]
        if sync:
            pltpu.sync_copy(srcsl, srcc[b])
            pltpu.sync_copy(dstsl, dstc[b])
        else:
            pltpu.async_copy(srcsl, srcc[b], isem[b])
            pltpu.async_copy(dstsl, dstc[b], isem[b])

    def _wait_idx(b, k):
        pltpu.make_async_copy(
            ei_hbm.at[0, pl.ds(_EC * (start + k), _EC)], srcc[b],
            isem[b]).wait()
        pltpu.make_async_copy(
            ei_hbm.at[1, pl.ds(_EC * (start + k), _EC)], dstc[b],
            isem[b]).wait()

    def _prime(b, k):
        for v in range(_EC // _LANES):
            sl = pl.ds(_LANES * v, _LANES)
            sv = srcc[b][sl]
            dv = dstc[b][sl]
            idx2[b][sl] = sv * 2 + c
            dix[b][sl] = dv
            s16 = sbase + sv * 8
            d16 = sbase + dv * 8 + 4
            for h in range(_HH):
                ixs[b][h][sl] = s16 + h
                jxs[b][h][sl] = d16 + h
        pltpu.async_copy(h2_hbm.at[idx2[b]], rows[b], gsem[b])
        for h in range(_HH):
            pltpu.async_copy(sflat_hbm.at[ixs[b][h]], gss[b][h], gsem[b])
            pltpu.async_copy(sflat_hbm.at[jxs[b][h]], gds[b][h], gsem[b])

    def _wait_gathers(b):
        pltpu.make_async_copy(h2_hbm.at[idx2[b]], rows[b], gsem[b]).wait()
        for h in range(_HH):
            pltpu.make_async_copy(
                sflat_hbm.at[ixs[b][h]], gss[b][h], gsem[b]).wait()
            pltpu.make_async_copy(
                sflat_hbm.at[jxs[b][h]], gds[b][h], gsem[b]).wait()

    def _process(b):
        _wait_gathers(b)
        for v in range(_EC // _LANES):
            sl = pl.ds(_LANES * v, _LANES)
            for h in range(_HH):
                a = gss[b][h][sl] + gds[b][h][sl]
                a = jnp.where(a >= 0.0, a, 0.2 * a)
                gds[b][h][sl] = jnp.exp(a)

        @pl.loop(0, _EC // _LANES)
        def _(g):
            ge = g * _LANES
            pr = [gds[b][h][pl.ds(ge, _LANES)] for h in range(_HH)]
            for i in range(_LANES):
                lane = jnp.full((_LANES,), i, jnp.int32)
                for h in range(_HH):
                    w = pr[h][lane]
                    sl0 = pl.ds(_D * h, _LANES)
                    sl1 = pl.ds(_D * h + _LANES, _LANES)
                    rows[b][ge + i, sl0] = rows[b][ge + i, sl0] * w
                    rows[b][ge + i, sl1] = rows[b][ge + i, sl1] * w

        pltpu.async_copy(rows[b], out_sp.at[dix[b]], ssem[b], add=True)
        for h in range(_HH):
            pltpu.async_copy(gds[b][h], asums[h].at[dix[b]], ssem[b],
                             add=True)

    def _wait_scatters(b):
        pltpu.make_async_copy(rows[b], out_sp.at[dix[b]], ssem[b]).wait()
        for h in range(_HH):
            pltpu.make_async_copy(
                gds[b][h], asums[h].at[dix[b]], ssem[b]).wait()

    _load_idx(0, jnp.int32(0), sync=True)
    _prime(0, jnp.int32(0))

    @pl.when(1 < count)
    def _():
        _load_idx(1, jnp.int32(1))

    @pl.loop(0, n_pairs)
    def _(kk):
        for b in (0, 1):
            k = 2 * kk + b
            nb_ = 1 - b

            @pl.when((k >= 1) & (k - 1 < count))
            def _():
                _wait_scatters(nb_)

            @pl.when(k + 1 < count)
            def _():
                _wait_idx(nb_, k + 1)
                _prime(nb_, k + 1)

            @pl.when(k + 2 < count)
            def _():
                _load_idx(b, k + 2)

            @pl.when(k < count)
            def _():
                _process(b)

    plsc.subcore_barrier()

    def _norm_block(n0, nrows):
        for h in range(_HH):
            pltpu.sync_copy(asums[h].at[pl.ds(n0, _FC)], abs_[h])

        @pl.loop(0, nrows // _LANES)
        def _(g):
            i0 = g * _LANES
            rec = [1.0 / (abs_[h][pl.ds(i0, _LANES)] + 1e-8)
                   for h in range(_HH)]
            for i in range(_LANES):
                lane = jnp.full((_LANES,), i, jnp.int32)
                for h in range(_HH):
                    w = rec[h][lane]
                    sl0 = pl.ds(_D * h, _LANES)
                    sl1 = pl.ds(_D * h + _LANES, _LANES)
                    fin[i0 + i, sl0] = fin[i0 + i, sl0] * w
                    fin[i0 + i, sl1] = fin[i0 + i, sl1] * w

    @pl.loop(0, n_rounds)
    def _(r):
        j = _NTILE * r + s

        @pl.when(j < n_full)
        def _():
            n0 = _FC * j
            pltpu.sync_copy(out_sp.at[pl.ds(n0, _FC)], fin)
            _norm_block(n0, _FC)
            pltpu.sync_copy(fin, out_hbm.at[c, pl.ds(n0, _FC)])

        if tail:
            @pl.when(j == n_full)
            def _():
                n0 = n_full * _FC
                pltpu.sync_copy(out_sp.at[pl.ds(n0, tail)],
                                fin.at[pl.ds(0, tail)])
                _norm_block(n0, tail)
                pltpu.sync_copy(fin.at[pl.ds(0, tail)],
                                out_hbm.at[c, pl.ds(n0, tail)])


def kernel(x, edge_index, W, b, a_src, a_dst, ln_gamma, ln_beta):
    n_nodes, nf = x.shape
    n_edges = edge_index.shape[1]
    n_pad = ((n_nodes + _FC - 1) // _FC) * _FC
    max_cnt = n_edges // _EC // _NTILE + 1

    blk = jnp.repeat(jnp.eye(_H, dtype=jnp.float32), _D, axis=0)
    asrc_m = blk * a_src.reshape(_HD, 1)
    adst_m = blk * a_dst.reshape(_HD, 1)
    acat = jnp.concatenate([asrc_m[:, :_HH], adst_m[:, :_HH],
                            asrc_m[:, _HH:], adst_m[:, _HH:]], axis=1)

    nb = 1000
    grid = n_nodes // nb
    h2, sall = pl.pallas_call(
        _tc_embed_kernel,
        grid=(grid,),
        in_specs=[
            pl.BlockSpec((nb, nf), lambda i: (i, 0)),
            pl.BlockSpec((nf, _HD), lambda i: (0, 0)),
            pl.BlockSpec((1, _HD), lambda i: (0, 0)),
            pl.BlockSpec((nf, 16), lambda i: (0, 0)),
        ],
        out_specs=[
            pl.BlockSpec((2 * nb, _HD // 2), lambda i: (i, 0)),
            pl.BlockSpec((2, nb, 8), lambda i: (0, i, 0)),
        ],
        out_shape=[
            jax.ShapeDtypeStruct((2 * n_nodes, _HD // 2), jnp.float32),
            jax.ShapeDtypeStruct((2, n_nodes, 8), jnp.float32),
        ],
    )(x, W, b.reshape(1, _HD), acat)

    sflat = sall.reshape(n_nodes * 16)

    mesh = plsc.VectorSubcoreMesh(
        core_axis_name="c", subcore_axis_name="s",
        num_cores=_NSC, num_subcores=_NTILE)
    scratch = (
        [pltpu.VMEM((_EC,), jnp.int32)] * 4
        + [pltpu.VMEM((_EC,), jnp.int32)] * 2
        + [pltpu.VMEM((_EC,), jnp.int32)] * (2 * _HH)
        + [pltpu.VMEM((_EC,), jnp.int32)] * (2 * _HH)
        + [pltpu.VMEM((_EC,), jnp.int32)] * 2
        + [pltpu.VMEM((_EC,), jnp.float32)] * (2 * _HH)
        + [pltpu.VMEM((_EC,), jnp.float32)] * (2 * _HH)
        + [pltpu.VMEM((_EC, _HD // 2), jnp.float32)] * 2
        + [pltpu.SemaphoreType.DMA] * 6
        + [pltpu.VMEM_SHARED((n_nodes, _HD // 2), jnp.float32)]
        + [pltpu.VMEM_SHARED((n_pad,), jnp.float32)] * _HH
    )
    out_sc = pl.kernel(
        functools.partial(_sc_edge_kernel, n_nodes, n_edges),
        out_type=jax.ShapeDtypeStruct((_NSC, n_nodes, _HD // 2), jnp.float32),
        mesh=mesh,
        scratch_types=scratch,
    )(h2, sflat, edge_index)

    out = pl.pallas_call(
        _tc_ln_kernel,
        grid=(grid,),
        in_specs=[
            pl.BlockSpec((_NSC, nb, _HD // 2), lambda i: (0, i, 0)),
            pl.BlockSpec((nb, nf), lambda i: (i, 0)),
            pl.BlockSpec((1, _HD), lambda i: (0, 0)),
            pl.BlockSpec((1, _HD), lambda i: (0, 0)),
        ],
        out_specs=pl.BlockSpec((nb, nf), lambda i: (i, 0)),
        out_shape=jax.ShapeDtypeStruct((n_nodes, nf), jnp.float32),
    )(out_sc, x, ln_gamma.reshape(1, _HD), ln_beta.reshape(1, _HD))
    return out
